# baseline (speedup 1.0000x reference)
import functools

import jax
import jax.numpy as jnp
from jax import lax
from jax.experimental import pallas as pl
from jax.experimental.pallas import tpu as pltpu
from jax.experimental.pallas import tpu_sc as plsc

_N_FIELDS = 26
_VOCAB = 100000
_EMB = 32
_B = 16384
_H1 = 512
_H2 = 256
_OUT = 1

_NIDX = _B * _N_FIELDS
_NW = 32
_BPW = _B // _NW
_WIN = 128
_NROW = _BPW // _WIN


def _sc_gather(tables, cat_data):
    mesh = plsc.VectorSubcoreMesh(core_axis_name="c", subcore_axis_name="s")

    @functools.partial(
        pl.kernel,
        out_type=jax.ShapeDtypeStruct((_NIDX, _EMB), jnp.bfloat16),
        mesh=mesh,
        scratch_types=[
            pltpu.VMEM((_BPW, _N_FIELDS), jnp.int32),
            pltpu.VMEM((_NROW, _WIN), jnp.int32),
            pltpu.VMEM((_NROW, _WIN), jnp.int32),
            pltpu.VMEM((_NROW, _WIN), jnp.int32),
            pltpu.VMEM((_BPW, _EMB), jnp.bfloat16),
            pltpu.VMEM((_BPW, _EMB), jnp.bfloat16),
            pltpu.SemaphoreType.DMA,
            pltpu.SemaphoreType.DMA,
        ],
        compiler_params=pltpu.CompilerParams(
            use_tc_tiling_on_sc=False, needs_layout_passes=False
        ),
    )
    def gather_kernel(tab_hbm, cat_hbm, out_hbm,
                      cat_v, idx_a, idx_b, oidx_v, rows_a, rows_b, gsem, ssem):
        wid = lax.axis_index("s") * 2 + lax.axis_index("c")
        b0 = wid * _BPW

        pltpu.sync_copy(cat_hbm.at[pl.ds(b0, _BPW)], cat_v)

        for j in range(_NROW):
            for v in range(0, _WIN, 16):
                lane = lax.iota(jnp.int32, 16) + (b0 + j * _WIN + v)
                oidx_v[j, pl.ds(v, 16)] = lane * _N_FIELDS

        def extract_idx(f, idx_v):
            for k in range(0, _BPW, 16):
                i0 = lax.iota(jnp.int32, 16) + k
                f16 = jnp.full((16,), f, jnp.int32)
                vals = plsc.load_gather(cat_v, [i0, f16])
                idx_v[k // _WIN, pl.ds(k % _WIN, 16)] = vals

        ibufs = (idx_a, idx_b)
        rbufs = (rows_a, rows_b)
        extract_idx(0, ibufs[0])
        scatters = []
        for f in range(_N_FIELDS):
            idx_v = ibufs[f % 2]
            rows_v = rbufs[f % 2]
            gathers = [
                pltpu.async_copy(
                    tab_hbm.at[f].at[idx_v.at[j]],
                    rows_v.at[pl.ds(j * _WIN, _WIN)],
                    gsem,
                )
                for j in range(_NROW)
            ]
            for c in scatters:
                c.wait()
            if f > 0:
                for j in range(_NROW):
                    for v in range(0, _WIN, 16):
                        oidx_v[j, pl.ds(v, 16)] = oidx_v[j, pl.ds(v, 16)] + 1
            if f + 1 < _N_FIELDS:
                extract_idx(f + 1, ibufs[(f + 1) % 2])
            for c in gathers:
                c.wait()
            scatters = [
                pltpu.async_copy(
                    rows_v.at[pl.ds(j * _WIN, _WIN)],
                    out_hbm.at[oidx_v.at[j]],
                    ssem,
                )
                for j in range(_NROW)
            ]
        for c in scatters:
            c.wait()

    return gather_kernel(tables, cat_data)


_BM = 1024


def _mlp_body(x_ref, w1_ref, b1_ref, w2_ref, b2_ref, w3_ref, b3_ref, o_ref):
    cdims = (((1,), (1,)), ((), ()))
    x = x_ref[...]
    h = lax.dot_general(x, w1_ref[...], cdims, preferred_element_type=jnp.float32)
    h = jnp.maximum(h + b1_ref[...], 0.0)
    h = lax.dot_general(h, w2_ref[...], cdims, preferred_element_type=jnp.float32)
    h = jnp.maximum(h + b2_ref[...], 0.0)
    o = jnp.sum(h * w3_ref[...], axis=1, keepdims=True)
    o = o + b3_ref[0, 0]
    o_ref[...] = jax.nn.sigmoid(o)


def _tc_mlp(x, W1, b1, W2, b2, W3, b3):
    n_embs = _N_FIELDS * _EMB
    grid = (_B // _BM,)
    return pl.pallas_call(
        _mlp_body,
        grid=grid,
        in_specs=[
            pl.BlockSpec((_BM, n_embs), lambda i: (i, 0)),
            pl.BlockSpec((_H1, n_embs), lambda i: (0, 0)),
            pl.BlockSpec((1, _H1), lambda i: (0, 0)),
            pl.BlockSpec((_H2, _H1), lambda i: (0, 0)),
            pl.BlockSpec((1, _H2), lambda i: (0, 0)),
            pl.BlockSpec((_OUT, _H2), lambda i: (0, 0)),
            pl.BlockSpec((1, _OUT), lambda i: (0, 0)),
        ],
        out_specs=pl.BlockSpec((_BM, _OUT), lambda i: (i, 0)),
        out_shape=jax.ShapeDtypeStruct((_B, _OUT), jnp.float32),
    )(x, W1, b1.reshape(1, _H1), W2, b2.reshape(1, _H2), W3, b3.reshape(1, _OUT))


def kernel(cat_data, tables, W1, b1, W2, b2, W3, b3):
    gathered = _sc_gather(tables.astype(jnp.bfloat16), cat_data)
    x = gathered.reshape(_B, _N_FIELDS * _EMB)
    return _tc_mlp(x, W1, b1, W2, b2, W3, b3)


# speedup vs baseline: 1.2346x; 1.2346x over previous
import functools

import jax
import jax.numpy as jnp
from jax import lax
from jax.experimental import pallas as pl
from jax.experimental.pallas import tpu as pltpu
from jax.experimental.pallas import tpu_sc as plsc

_N_FIELDS = 26
_VOCAB = 100000
_EMB = 32
_B = 16384
_H1 = 512
_H2 = 256
_OUT = 1

_NIDX = _B * _N_FIELDS
_NW = 32
_BPW = _B // _NW
_WIN = 128
_NROW = _BPW // _WIN


def _sc_gather(tables, cat_data):
    mesh = plsc.VectorSubcoreMesh(core_axis_name="c", subcore_axis_name="s")

    @functools.partial(
        pl.kernel,
        out_type=jax.ShapeDtypeStruct((_NIDX, _EMB), jnp.float32),
        mesh=mesh,
        scratch_types=[
            pltpu.VMEM((_BPW, _N_FIELDS), jnp.int32),
            pltpu.VMEM((_NROW, _WIN), jnp.int32),
            pltpu.VMEM((_NROW, _WIN), jnp.int32),
            pltpu.VMEM((_NROW, _WIN), jnp.int32),
            pltpu.VMEM((_BPW, _EMB), jnp.float32),
            pltpu.VMEM((_BPW, _EMB), jnp.float32),
            pltpu.SemaphoreType.DMA,
            pltpu.SemaphoreType.DMA,
        ],
        compiler_params=pltpu.CompilerParams(
            use_tc_tiling_on_sc=False, needs_layout_passes=False
        ),
    )
    def gather_kernel(tab_hbm, cat_hbm, out_hbm,
                      cat_v, idx_a, idx_b, oidx_v, rows_a, rows_b, gsem, ssem):
        wid = lax.axis_index("s") * 2 + lax.axis_index("c")
        b0 = wid * _BPW

        pltpu.sync_copy(cat_hbm.at[pl.ds(b0, _BPW)], cat_v)

        for j in range(_NROW):
            for v in range(0, _WIN, 16):
                lane = lax.iota(jnp.int32, 16) + (b0 + j * _WIN + v)
                oidx_v[j, pl.ds(v, 16)] = lane * _N_FIELDS

        def extract_idx(f, idx_v):
            for k in range(0, _BPW, 16):
                i0 = lax.iota(jnp.int32, 16) + k
                f16 = jnp.full((16,), f, jnp.int32)
                vals = plsc.load_gather(cat_v, [i0, f16])
                idx_v[k // _WIN, pl.ds(k % _WIN, 16)] = vals

        ibufs = (idx_a, idx_b)
        rbufs = (rows_a, rows_b)
        extract_idx(0, ibufs[0])
        scatters = []
        for f in range(_N_FIELDS):
            idx_v = ibufs[f % 2]
            rows_v = rbufs[f % 2]
            gathers = [
                pltpu.async_copy(
                    tab_hbm.at[f].at[idx_v.at[j]],
                    rows_v.at[pl.ds(j * _WIN, _WIN)],
                    gsem,
                )
                for j in range(_NROW)
            ]
            for c in scatters:
                c.wait()
            if f > 0:
                for j in range(_NROW):
                    for v in range(0, _WIN, 16):
                        oidx_v[j, pl.ds(v, 16)] = oidx_v[j, pl.ds(v, 16)] + 1
            if f + 1 < _N_FIELDS:
                extract_idx(f + 1, ibufs[(f + 1) % 2])
            for c in gathers:
                c.wait()
            scatters = [
                pltpu.async_copy(
                    rows_v.at[pl.ds(j * _WIN, _WIN)],
                    out_hbm.at[oidx_v.at[j]],
                    ssem,
                )
                for j in range(_NROW)
            ]
        for c in scatters:
            c.wait()

    return gather_kernel(tables, cat_data)


_BM = 1024


def _mlp_body(x_ref, w1_ref, b1_ref, w2_ref, b2_ref, w3_ref, b3_ref, o_ref):
    cdims = (((1,), (1,)), ((), ()))
    x = x_ref[...]
    h = lax.dot_general(x, w1_ref[...], cdims, preferred_element_type=jnp.float32)
    h = jnp.maximum(h + b1_ref[...], 0.0)
    h = lax.dot_general(h, w2_ref[...], cdims, preferred_element_type=jnp.float32)
    h = jnp.maximum(h + b2_ref[...], 0.0)
    o = jnp.sum(h * w3_ref[...], axis=1, keepdims=True)
    o = o + b3_ref[0, 0]
    o_ref[...] = jax.nn.sigmoid(o)


def _tc_mlp(x, W1, b1, W2, b2, W3, b3):
    n_embs = _N_FIELDS * _EMB
    grid = (_B // _BM,)
    return pl.pallas_call(
        _mlp_body,
        grid=grid,
        in_specs=[
            pl.BlockSpec((_BM, n_embs), lambda i: (i, 0)),
            pl.BlockSpec((_H1, n_embs), lambda i: (0, 0)),
            pl.BlockSpec((1, _H1), lambda i: (0, 0)),
            pl.BlockSpec((_H2, _H1), lambda i: (0, 0)),
            pl.BlockSpec((1, _H2), lambda i: (0, 0)),
            pl.BlockSpec((_OUT, _H2), lambda i: (0, 0)),
            pl.BlockSpec((1, _OUT), lambda i: (0, 0)),
        ],
        out_specs=pl.BlockSpec((_BM, _OUT), lambda i: (i, 0)),
        out_shape=jax.ShapeDtypeStruct((_B, _OUT), jnp.float32),
    )(x, W1, b1.reshape(1, _H1), W2, b2.reshape(1, _H2), W3, b3.reshape(1, _OUT))


def kernel(cat_data, tables, W1, b1, W2, b2, W3, b3):
    gathered = _sc_gather(tables, cat_data)
    x = gathered.reshape(_B, _N_FIELDS * _EMB)
    return _tc_mlp(x, W1, b1, W2, b2, W3, b3)


# speedup vs baseline: 1.2440x; 1.0076x over previous
import functools

import jax
import jax.numpy as jnp
from jax import lax
from jax.experimental import pallas as pl
from jax.experimental.pallas import tpu as pltpu
from jax.experimental.pallas import tpu_sc as plsc

_N_FIELDS = 26
_VOCAB = 100000
_EMB = 32
_B = 16384
_H1 = 512
_H2 = 256
_OUT = 1

_NIDX = _B * _N_FIELDS
_NW = 32
_BPW = _B // _NW
_WIN = 128
_NROW = _BPW // _WIN


def _sc_gather(tables, cat_data, nb):
    mesh = plsc.VectorSubcoreMesh(core_axis_name="c", subcore_axis_name="s")
    bpw = nb // _NW
    nrow = bpw // _WIN

    @functools.partial(
        pl.kernel,
        out_type=jax.ShapeDtypeStruct((nb * _N_FIELDS, _EMB), jnp.float32),
        mesh=mesh,
        scratch_types=[
            pltpu.VMEM((bpw, _N_FIELDS), jnp.int32),
            pltpu.VMEM((nrow, _WIN), jnp.int32),
            pltpu.VMEM((nrow, _WIN), jnp.int32),
            pltpu.VMEM((nrow, _WIN), jnp.int32),
            pltpu.VMEM((bpw, _EMB), jnp.float32),
            pltpu.VMEM((bpw, _EMB), jnp.float32),
            pltpu.SemaphoreType.DMA,
            pltpu.SemaphoreType.DMA,
        ],
        compiler_params=pltpu.CompilerParams(
            use_tc_tiling_on_sc=False, needs_layout_passes=False
        ),
    )
    def gather_kernel(tab_hbm, cat_hbm, out_hbm,
                      cat_v, idx_a, idx_b, oidx_v, rows_a, rows_b, gsem, ssem):
        wid = lax.axis_index("s") * 2 + lax.axis_index("c")
        b0 = wid * bpw

        pltpu.sync_copy(cat_hbm.at[pl.ds(b0, bpw)], cat_v)

        for j in range(nrow):
            for v in range(0, _WIN, 16):
                lane = lax.iota(jnp.int32, 16) + (b0 + j * _WIN + v)
                oidx_v[j, pl.ds(v, 16)] = lane * _N_FIELDS

        def extract_idx(f, idx_v):
            for k in range(0, bpw, 16):
                i0 = lax.iota(jnp.int32, 16) + k
                f16 = jnp.full((16,), f, jnp.int32)
                vals = plsc.load_gather(cat_v, [i0, f16])
                idx_v[k // _WIN, pl.ds(k % _WIN, 16)] = vals

        ibufs = (idx_a, idx_b)
        rbufs = (rows_a, rows_b)
        extract_idx(0, ibufs[0])
        scatters = []
        for f in range(_N_FIELDS):
            idx_v = ibufs[f % 2]
            rows_v = rbufs[f % 2]
            gathers = [
                pltpu.async_copy(
                    tab_hbm.at[f].at[idx_v.at[j]],
                    rows_v.at[pl.ds(j * _WIN, _WIN)],
                    gsem,
                )
                for j in range(nrow)
            ]
            for c in scatters:
                c.wait()
            if f > 0:
                for j in range(nrow):
                    for v in range(0, _WIN, 16):
                        oidx_v[j, pl.ds(v, 16)] = oidx_v[j, pl.ds(v, 16)] + 1
            if f + 1 < _N_FIELDS:
                extract_idx(f + 1, ibufs[(f + 1) % 2])
            for c in gathers:
                c.wait()
            scatters = [
                pltpu.async_copy(
                    rows_v.at[pl.ds(j * _WIN, _WIN)],
                    out_hbm.at[oidx_v.at[j]],
                    ssem,
                )
                for j in range(nrow)
            ]
        for c in scatters:
            c.wait()

    return gather_kernel(tables, cat_data)


_BM = 1024


def _mlp_body(x_ref, w1_ref, b1_ref, w2_ref, b2_ref, w3_ref, b3_ref, o_ref):
    cdims = (((1,), (1,)), ((), ()))
    x = x_ref[...]
    h = lax.dot_general(x, w1_ref[...], cdims, preferred_element_type=jnp.float32)
    h = jnp.maximum(h + b1_ref[...], 0.0)
    h = lax.dot_general(h, w2_ref[...], cdims, preferred_element_type=jnp.float32)
    h = jnp.maximum(h + b2_ref[...], 0.0)
    o = jnp.sum(h * w3_ref[...], axis=1, keepdims=True)
    o = o + b3_ref[0, 0]
    o_ref[...] = jax.nn.sigmoid(o)


def _tc_mlp(x, W1, b1, W2, b2, W3, b3):
    n_embs = _N_FIELDS * _EMB
    nb = x.shape[0]
    grid = (nb // _BM,)
    return pl.pallas_call(
        _mlp_body,
        grid=grid,
        in_specs=[
            pl.BlockSpec((_BM, n_embs), lambda i: (i, 0)),
            pl.BlockSpec((_H1, n_embs), lambda i: (0, 0)),
            pl.BlockSpec((1, _H1), lambda i: (0, 0)),
            pl.BlockSpec((_H2, _H1), lambda i: (0, 0)),
            pl.BlockSpec((1, _H2), lambda i: (0, 0)),
            pl.BlockSpec((_OUT, _H2), lambda i: (0, 0)),
            pl.BlockSpec((1, _OUT), lambda i: (0, 0)),
        ],
        out_specs=pl.BlockSpec((_BM, _OUT), lambda i: (i, 0)),
        out_shape=jax.ShapeDtypeStruct((nb, _OUT), jnp.float32),
    )(x, W1, b1.reshape(1, _H1), W2, b2.reshape(1, _H2), W3, b3.reshape(1, _OUT))


def kernel(cat_data, tables, W1, b1, W2, b2, W3, b3):
    hb = _B // 2
    outs = []
    for h in range(2):
        cat_h = jax.lax.slice_in_dim(cat_data, h * hb, (h + 1) * hb, axis=0)
        gathered = _sc_gather(tables, cat_h, hb)
        x = gathered.reshape(hb, _N_FIELDS * _EMB)
        outs.append(_tc_mlp(x, W1, b1, W2, b2, W3, b3))
    return jnp.concatenate(outs, axis=0)
